# baseline (speedup 1.0000x reference)
import jax
import jax.numpy as jnp
from jax.experimental import pallas as pl
from jax.experimental.pallas import tpu as pltpu

_RO = 32
_NCHUNK = 128 // _RO
_GRP = 16


def _bf(a):
    return a.astype(jnp.bfloat16).astype(jnp.float32)


def _shift_r(a):
    z = jnp.zeros(a.shape[:-1] + (1,), a.dtype)
    return jnp.concatenate([z, a[..., :-1]], axis=-1)


def _shift_d_zero(a):
    z = jnp.zeros((1, a.shape[-1]), a.dtype)
    return jnp.concatenate([z, a[:-1, :]], axis=-2)


def _shift_l(a):
    z = jnp.zeros(a.shape[:-1] + (1,), a.dtype)
    return jnp.concatenate([a[..., 1:], z], axis=-1)


def _perm_eo(a):
    perm = jax.lax.broadcasted_iota(jnp.int32, a.shape, a.ndim - 1)
    perm = (perm % 64) * 2 + perm // 64
    return jnp.take_along_axis(a, perm, axis=-1)


def _stitch(left, right):
    e = jnp.concatenate([left[..., :64], right[..., :64]], axis=-1)
    o = jnp.concatenate([left[..., 64:], right[..., 64:]], axis=-1)
    return e, o


def _load_parity(lref, rref, r0, nrows, roff):
    lo = roff + 2 * r0
    hi = lo + 2 * nrows
    pl_ = _perm_eo(_bf(lref[slice(lo, hi, 2), slice(None)]))
    pr_ = _perm_eo(_bf(rref[slice(lo, hi, 2), slice(None)]))
    return _stitch(pl_, pr_)


def _pool_body(xl_ref, xr_ref, dl_ref, dr_ref, o_ref, ws_ref, xs_ref):
    CB = xl_ref.shape[1]

    @pl.when(pl.program_id(1) == 0)
    def _weights():
        _weights_phase(dl_ref, dr_ref, ws_ref)

    _x_phases(xl_ref, xr_ref, o_ref, ws_ref, xs_ref, CB)


def _weights_phase(dl_ref, dr_ref, ws_ref):
    for ci in range(_NCHUNK):
        r0 = ci * _RO
        dl = dl_ref.at[0, 0]
        dr = dr_ref.at[0, 0]
        d0, deo = _load_parity(dl, dr, r0, _RO, 0)
        doe, doo = _load_parity(dl, dr, r0, _RO, 1)
        if ci == 0:
            n_oe = _shift_d_zero(doe)
            n_oo = _shift_d_zero(doo)
        else:
            n_oe, n_oo = _load_parity(dl, dr, r0, _RO, -1)

        col_ok = jax.lax.broadcasted_iota(jnp.int32, (_RO, 128), 1) > 0
        cv = jnp.where(col_ok, 3.0, 2.0)
        if ci == 0:
            row_ok = jax.lax.broadcasted_iota(jnp.int32, (_RO, 128), 0) > 0
            rv = jnp.where(row_ok, 3.0, 2.0)
        else:
            row_ok = None
            rv = 3.0
        inv = 1.0 / (rv * cv)

        def w(dv):
            return jnp.exp(-jnp.abs(dv - d0)) * inv

        def wmask(wv, rmask, cmask):
            m = None
            if rmask is not None and cmask is not None:
                m = rmask & cmask
            elif rmask is not None:
                m = rmask
            elif cmask is not None:
                m = cmask
            return jnp.where(m, wv, 0.0) if m is not None else wv

        rows = slice(r0, r0 + _RO)
        ws_ref[0, rows, :] = inv
        ws_ref[1, rows, :] = w(deo)
        ws_ref[3, rows, :] = w(doe)
        ws_ref[4, rows, :] = wmask(w(n_oe), row_ok, None)
        ws_ref[5, rows, :] = w(doo)
        ws_ref[7, rows, :] = wmask(w(n_oo), row_ok, None)
        ws_ref[2, rows, :] = _shift_l(
            wmask(w(_shift_r(deo)), None, col_ok))
        ws_ref[6, rows, :] = _shift_l(
            wmask(w(_shift_r(doo)), None, col_ok))
        ws_ref[8, rows, :] = _shift_l(
            wmask(w(_shift_r(n_oo)), row_ok, col_ok))


def _x_phases(xl_ref, xr_ref, o_ref, ws_ref, xs_ref, CB):
    for g in range(0, CB, _GRP):
        gs = slice(g, g + _GRP)
        for roff, (ie, io) in ((0, (0, 1)), (1, (2, 3))):
            pL = _perm_eo(_bf(xl_ref[0, gs, slice(roff, roff + 256, 2), :]))
            pR = _perm_eo(_bf(xr_ref[0, gs, slice(roff, roff + 256, 2), :]))
            xs_ref[ie, gs, 1:129, 0:64] = pL[..., 0:64]
            xs_ref[ie, gs, 1:129, 64:128] = pR[..., 0:64]
            xs_ref[io, gs, 1:129, 0:64] = pL[..., 64:128]
            xs_ref[io, gs, 1:129, 64:128] = pR[..., 64:128]
        xs_ref[2, gs, 0:1, :] = jnp.zeros((_GRP, 1, 128), jnp.float32)
        xs_ref[3, gs, 0:1, :] = jnp.zeros((_GRP, 1, 128), jnp.float32)

    for ci in range(_NCHUNK):
        r0 = ci * _RO
        rows = slice(r0, r0 + _RO)
        r_c = slice(r0 + 1, r0 + 1 + _RO)
        r_p = slice(r0, r0 + _RO)
        w0 = ws_ref[0, rows, :]
        w1 = ws_ref[1, rows, :]
        w2 = ws_ref[2, rows, :]
        w3 = ws_ref[3, rows, :]
        w4 = ws_ref[4, rows, :]
        w5 = ws_ref[5, rows, :]
        w6 = ws_ref[6, rows, :]
        w7 = ws_ref[7, rows, :]
        w8 = ws_ref[8, rows, :]
        for c in range(CB):
            xee = xs_ref[0, c, r_c, :]
            xeo = xs_ref[1, c, r_c, :]
            xoe = xs_ref[2, c, r_c, :]
            sd_oe = xs_ref[2, c, r_p, :]
            xoo = xs_ref[3, c, r_c, :]
            sd_oo = xs_ref[3, c, r_p, :]
            acc = (w0 * xee
                   + w1 * xeo
                   + w3 * xoe
                   + w4 * sd_oe
                   + w5 * xoo
                   + w7 * sd_oo
                   + _shift_r(w2 * xeo + w6 * xoo + w8 * sd_oo))
            o_ref[0, c, rows, :] = acc


def kernel(input, depth):
    B, C, H, W = input.shape
    CB = 32
    grid = (B, C // CB)
    Wh = W // 2
    return pl.pallas_call(
        _pool_body,
        grid=grid,
        in_specs=[
            pl.BlockSpec((1, CB, H, Wh), lambda b, c: (b, c, 0, 0)),
            pl.BlockSpec((1, CB, H, Wh), lambda b, c: (b, c, 0, 1)),
            pl.BlockSpec((1, 1, H, Wh), lambda b, c: (b, 0, 0, 0)),
            pl.BlockSpec((1, 1, H, Wh), lambda b, c: (b, 0, 0, 1)),
        ],
        out_specs=pl.BlockSpec((1, CB, H // 2, W // 2),
                               lambda b, c: (b, c, 0, 0)),
        out_shape=jax.ShapeDtypeStruct((B, C, H // 2, W // 2), input.dtype),
        scratch_shapes=[
            pltpu.VMEM((9, 128, 128), jnp.float32),
            pltpu.VMEM((4, CB, 136, 128), jnp.float32),
        ],
        compiler_params=pltpu.CompilerParams(
            dimension_semantics=("parallel", "parallel"),
            vmem_limit_bytes=100 * 1024 * 1024,
        ),
    )(input, input, depth, depth)


# speedup vs baseline: 1.0685x; 1.0685x over previous
import jax
import jax.numpy as jnp
from jax.experimental import pallas as pl
from jax.experimental.pallas import tpu as pltpu

_RO = 32
_NCHUNK = 128 // _RO
_GRP = 16


def _bf(a):
    return a.astype(jnp.bfloat16).astype(jnp.float32)


def _shift_r(a):
    z = jnp.zeros(a.shape[:-1] + (1,), a.dtype)
    return jnp.concatenate([z, a[..., :-1]], axis=-1)


def _shift_d_zero(a):
    z = jnp.zeros((1, a.shape[-1]), a.dtype)
    return jnp.concatenate([z, a[:-1, :]], axis=-2)


def _shift_l(a):
    z = jnp.zeros(a.shape[:-1] + (1,), a.dtype)
    return jnp.concatenate([a[..., 1:], z], axis=-1)


def _perm_eo(a):
    perm = jax.lax.broadcasted_iota(jnp.int32, a.shape, a.ndim - 1)
    perm = (perm % 64) * 2 + perm // 64
    return jnp.take_along_axis(a, perm, axis=-1)


def _stitch(left, right):
    e = jnp.concatenate([left[..., :64], right[..., :64]], axis=-1)
    o = jnp.concatenate([left[..., 64:], right[..., 64:]], axis=-1)
    return e, o


def _load_parity(lref, rref, r0, nrows, roff):
    lo = roff + 2 * r0
    hi = lo + 2 * nrows
    pl_ = _perm_eo(_bf(lref[slice(lo, hi, 2), slice(None)]))
    pr_ = _perm_eo(_bf(rref[slice(lo, hi, 2), slice(None)]))
    return _stitch(pl_, pr_)


def _pool_body(xl_ref, xr_ref, dl_ref, dr_ref, o_ref, ws_ref, xs_ref):
    CB = xl_ref.shape[1]

    @pl.when(pl.program_id(1) == 0)
    def _weights():
        _weights_phase(dl_ref, dr_ref, ws_ref)

    _x_phases(xl_ref, xr_ref, o_ref, ws_ref, xs_ref, CB)


def _weights_phase(dl_ref, dr_ref, ws_ref):
    for ci in range(_NCHUNK):
        r0 = ci * _RO
        dl = dl_ref.at[0, 0]
        dr = dr_ref.at[0, 0]
        d0, deo = _load_parity(dl, dr, r0, _RO, 0)
        doe, doo = _load_parity(dl, dr, r0, _RO, 1)
        if ci == 0:
            n_oe = _shift_d_zero(doe)
            n_oo = _shift_d_zero(doo)
        else:
            n_oe, n_oo = _load_parity(dl, dr, r0, _RO, -1)

        col_ok = jax.lax.broadcasted_iota(jnp.int32, (_RO, 128), 1) > 0
        cv = jnp.where(col_ok, 3.0, 2.0)
        if ci == 0:
            row_ok = jax.lax.broadcasted_iota(jnp.int32, (_RO, 128), 0) > 0
            rv = jnp.where(row_ok, 3.0, 2.0)
        else:
            row_ok = None
            rv = 3.0
        inv = 1.0 / (rv * cv)

        def w(dv):
            return jnp.exp(-jnp.abs(dv - d0)) * inv

        def wmask(wv, rmask, cmask):
            m = None
            if rmask is not None and cmask is not None:
                m = rmask & cmask
            elif rmask is not None:
                m = rmask
            elif cmask is not None:
                m = cmask
            return jnp.where(m, wv, 0.0) if m is not None else wv

        rows = slice(r0, r0 + _RO)
        ws_ref[0, rows, :] = inv
        ws_ref[1, rows, :] = w(deo)
        ws_ref[3, rows, :] = w(doe)
        ws_ref[4, rows, :] = wmask(w(n_oe), row_ok, None)
        ws_ref[5, rows, :] = w(doo)
        ws_ref[7, rows, :] = wmask(w(n_oo), row_ok, None)
        ws_ref[2, rows, :] = _shift_l(
            wmask(w(_shift_r(deo)), None, col_ok))
        ws_ref[6, rows, :] = _shift_l(
            wmask(w(_shift_r(doo)), None, col_ok))
        ws_ref[8, rows, :] = _shift_l(
            wmask(w(_shift_r(n_oo)), row_ok, col_ok))


def _x_phases(xl_ref, xr_ref, o_ref, ws_ref, xs_ref, CB):
    for g in range(0, CB, _GRP):
        gs = slice(g, g + _GRP)
        for roff, (ie, io) in ((0, (0, 1)), (1, (2, 3))):
            pL = _perm_eo(_bf(xl_ref[0, gs, slice(roff, roff + 256, 2), :]))
            pR = _perm_eo(_bf(xr_ref[0, gs, slice(roff, roff + 256, 2), :]))
            xs_ref[ie, gs, 1:129, 0:64] = pL[..., 0:64]
            xs_ref[ie, gs, 1:129, 64:128] = pR[..., 0:64]
            xs_ref[io, gs, 1:129, 0:64] = pL[..., 64:128]
            xs_ref[io, gs, 1:129, 64:128] = pR[..., 64:128]
        xs_ref[2, gs, 0:1, :] = jnp.zeros((_GRP, 1, 128), jnp.float32)
        xs_ref[3, gs, 0:1, :] = jnp.zeros((_GRP, 1, 128), jnp.float32)

    for ci in range(_NCHUNK):
        r0 = ci * _RO
        rows = slice(r0, r0 + _RO)
        r_c = slice(r0 + 1, r0 + 1 + _RO)
        r_p = slice(r0, r0 + _RO)
        w0 = ws_ref[0, rows, :]
        w1 = ws_ref[1, rows, :]
        w2 = ws_ref[2, rows, :]
        w3 = ws_ref[3, rows, :]
        w4 = ws_ref[4, rows, :]
        w5 = ws_ref[5, rows, :]
        w6 = ws_ref[6, rows, :]
        w7 = ws_ref[7, rows, :]
        w8 = ws_ref[8, rows, :]
        for c in range(CB):
            xee = xs_ref[0, c, r_c, :]
            xeo = xs_ref[1, c, r_c, :]
            xoe = xs_ref[2, c, r_c, :]
            sd_oe = xs_ref[2, c, r_p, :]
            xoo = xs_ref[3, c, r_c, :]
            sd_oo = xs_ref[3, c, r_p, :]
            acc = (w0 * xee
                   + w1 * xeo
                   + w3 * xoe
                   + w4 * sd_oe
                   + w5 * xoo
                   + w7 * sd_oo
                   + _shift_r(w2 * xeo + w6 * xoo + w8 * sd_oo))
            o_ref[0, c, rows, :] = acc


def kernel(input, depth):
    B, C, H, W = input.shape
    CB = 64
    grid = (B, C // CB)
    Wh = W // 2
    return pl.pallas_call(
        _pool_body,
        grid=grid,
        in_specs=[
            pl.BlockSpec((1, CB, H, Wh), lambda b, c: (b, c, 0, 0)),
            pl.BlockSpec((1, CB, H, Wh), lambda b, c: (b, c, 0, 1)),
            pl.BlockSpec((1, 1, H, Wh), lambda b, c: (b, 0, 0, 0)),
            pl.BlockSpec((1, 1, H, Wh), lambda b, c: (b, 0, 0, 1)),
        ],
        out_specs=pl.BlockSpec((1, CB, H // 2, W // 2),
                               lambda b, c: (b, c, 0, 0)),
        out_shape=jax.ShapeDtypeStruct((B, C, H // 2, W // 2), input.dtype),
        scratch_shapes=[
            pltpu.VMEM((9, 128, 128), jnp.float32),
            pltpu.VMEM((4, CB, 136, 128), jnp.float32),
        ],
        compiler_params=pltpu.CompilerParams(
            dimension_semantics=("parallel", "parallel"),
            vmem_limit_bytes=100 * 1024 * 1024,
        ),
    )(input, input, depth, depth)
